# Initial kernel scaffold; baseline (speedup 1.0000x reference)
import jax
import jax.numpy as jnp
from jax.experimental import pallas as pl


def kernel(node_states, adj_list_0, adj_list_1, adj_list_2, adj_list_3, W_0_0, W_0_1, W_1_0, W_1_1, W_2_0, W_2_1, W_3_0, W_3_1):
    raise NotImplementedError("write your pallas kernel here")


# speedup vs baseline: 2.5295x; 2.5295x over previous
import functools

import jax
import jax.numpy as jnp
from jax import lax
from jax.experimental import pallas as pl
from jax.experimental.pallas import tpu as pltpu
from jax.experimental.pallas import tpu_sc as plsc

N = 10000
D = 128
E = 80000
T = 4

NC = 2
NS = 16
NW = NC * NS

HIGH = jax.lax.Precision.HIGHEST


def _precompute_body(ns_ref, w_ref, ab_ref):
    x = ns_ref[...]
    w = w_ref[0]
    y = jnp.dot(x, w, precision=HIGH, preferred_element_type=jnp.float32)
    ab_ref[0, 0] = y[:, :D]
    ab_ref[1, 0] = y[:, D:]


def _precompute(ns, w0cat):
    BN = 1000
    return pl.pallas_call(
        _precompute_body,
        grid=(T, N // BN),
        in_specs=[
            pl.BlockSpec((BN, D), lambda t, i: (i, 0)),
            pl.BlockSpec((1, D, 2 * D), lambda t, i: (t, 0, 0)),
        ],
        out_specs=pl.BlockSpec((2, 1, BN, D), lambda t, i: (0, t, i, 0)),
        out_shape=jax.ShapeDtypeStruct((2, T, N, D), jnp.float32),
    )(ns, w0cat)


def _gather_rows(table, gidx):
    G = gidx.shape[0]
    per_w = G // NW
    C = 200
    steps = per_w // C
    mesh = plsc.VectorSubcoreMesh(core_axis_name="c", subcore_axis_name="s")

    @functools.partial(
        pl.kernel,
        out_type=jax.ShapeDtypeStruct((G, D), jnp.float32),
        mesh=mesh,
        scratch_types=[
            pltpu.VMEM((C,), jnp.int32),
            pltpu.VMEM((C, D), jnp.float32),
            pltpu.SemaphoreType.DMA,
        ],
    )
    def k(table_hbm, idx_hbm, out_hbm, idx_v, rows_v, sem):
        wid = lax.axis_index("s") * NC + lax.axis_index("c")
        base = wid * per_w

        @pl.loop(0, steps)
        def _(i):
            off = base + i * C
            pltpu.sync_copy(idx_hbm.at[pl.ds(off, C)], idx_v)
            pltpu.async_copy(table_hbm.at[idx_v], rows_v, sem).wait()
            pltpu.sync_copy(rows_v, out_hbm.at[pl.ds(off, C)])

    return k(table, gidx)


def _mlp_body(a_ref, b_ref, w_ref, m_ref):
    x = jnp.maximum(a_ref[0] + b_ref[0], 0.0)
    m = jnp.dot(x, w_ref[0], precision=HIGH, preferred_element_type=jnp.float32)
    m_ref[...] = jnp.maximum(m, 0.0)


def _mlp(g2, w1s):
    BE = 1000
    per_t = E // BE
    return pl.pallas_call(
        _mlp_body,
        grid=(T, per_t),
        in_specs=[
            pl.BlockSpec((1, BE, D), lambda t, i: (0, t * per_t + i, 0)),
            pl.BlockSpec((1, BE, D), lambda t, i: (1, t * per_t + i, 0)),
            pl.BlockSpec((1, D, D), lambda t, i: (t, 0, 0)),
        ],
        out_specs=pl.BlockSpec((BE, D), lambda t, i: (t * per_t + i, 0)),
        out_shape=jax.ShapeDtypeStruct((T * E, D), jnp.float32),
    )(g2, g2, w1s)


def _scatter_add(msgs, tgts, zeros):
    M = msgs.shape[0]
    per_w = M // NW
    C = 200
    steps = per_w // C
    rows_per_tile = 624
    tail = N - NS * rows_per_tile
    mesh = plsc.VectorSubcoreMesh(core_axis_name="c", subcore_axis_name="s")

    @functools.partial(
        pl.kernel,
        out_type=jax.ShapeDtypeStruct((NC, N, D), jnp.float32),
        mesh=mesh,
        scratch_types=[
            pltpu.VMEM((C,), jnp.int32),
            pltpu.VMEM((C, D), jnp.float32),
            pltpu.VMEM_SHARED((N, D), jnp.float32),
            pltpu.SemaphoreType.DMA,
        ],
    )
    def k(msg_hbm, tgt_hbm, zero_hbm, out_hbm, idx_v, m_v, acc, sem):
        cid = lax.axis_index("c")
        sid = lax.axis_index("s")
        wid = sid * NC + cid
        base = wid * per_w

        @pl.when(sid == 0)
        def _():
            pltpu.sync_copy(zero_hbm, acc)

        plsc.subcore_barrier()

        @pl.loop(0, steps)
        def _(i):
            off = base + i * C
            pltpu.sync_copy(tgt_hbm.at[pl.ds(off, C)], idx_v)
            pltpu.async_copy(msg_hbm.at[pl.ds(off, C)], m_v, sem).wait()
            pltpu.sync_copy(m_v, acc.at[idx_v], add=True)

        plsc.subcore_barrier()
        r0 = sid * rows_per_tile
        pltpu.sync_copy(acc.at[pl.ds(r0, rows_per_tile)],
                        out_hbm.at[cid, pl.ds(r0, rows_per_tile)])

        @pl.when(sid == 0)
        def _():
            t0 = NS * rows_per_tile
            pltpu.sync_copy(acc.at[pl.ds(t0, tail)],
                            out_hbm.at[cid, pl.ds(t0, tail)])

    return k(msgs, tgts, zeros)


def _sum_partials_body(p_ref, o_ref):
    o_ref[...] = p_ref[0] + p_ref[1]


def _sum_partials(partials):
    BN = 1000
    return pl.pallas_call(
        _sum_partials_body,
        grid=(N // BN,),
        in_specs=[pl.BlockSpec((2, BN, D), lambda i: (0, i, 0))],
        out_specs=pl.BlockSpec((BN, D), lambda i: (i, 0)),
        out_shape=jax.ShapeDtypeStruct((N, D), jnp.float32),
    )(partials)


def kernel(node_states, adj_list_0, adj_list_1, adj_list_2, adj_list_3,
           W_0_0, W_0_1, W_1_0, W_1_1, W_2_0, W_2_1, W_3_0, W_3_1):
    adj = [adj_list_0, adj_list_1, adj_list_2, adj_list_3]
    w0cat = (jnp.stack([W_0_0, W_1_0, W_2_0, W_3_0])
             .reshape(T, 2, D, D).transpose(0, 2, 1, 3).reshape(T, D, 2 * D))
    w1s = jnp.stack([W_0_1, W_1_1, W_2_1, W_3_1])

    srcs = jnp.concatenate([adj[t][:, 0] + t * N for t in range(T)])
    tgts = jnp.concatenate([adj[t][:, 1] for t in range(T)])
    gidx = jnp.concatenate([srcs, tgts + T * N + jnp.repeat(
        jnp.arange(T, dtype=jnp.int32) * N, E)])

    ab = _precompute(node_states, w0cat)
    table = ab.reshape(2 * T * N, D)
    g = _gather_rows(table, gidx)
    msgs = _mlp(g.reshape(2, T * E, D), w1s)
    zeros = jnp.zeros((N, D), jnp.float32)
    partials = _scatter_add(msgs, tgts, zeros)
    return _sum_partials(partials)


# speedup vs baseline: 2.8456x; 1.1250x over previous
import functools

import jax
import jax.numpy as jnp
from jax import lax
from jax.experimental import pallas as pl
from jax.experimental.pallas import tpu as pltpu
from jax.experimental.pallas import tpu_sc as plsc

N = 10000
D = 128
E = 80000
T = 4

NC = 2
NS = 16
NW = NC * NS

HIGH = jax.lax.Precision.DEFAULT


def _precompute_body(ns_ref, w_ref, ab_ref):
    x = ns_ref[...]
    w = w_ref[0]
    y = jnp.dot(x, w, precision=HIGH, preferred_element_type=jnp.float32)
    ab_ref[0, 0] = y[:, :D]
    ab_ref[1, 0] = y[:, D:]


def _precompute(ns, w0cat):
    BN = 1000
    return pl.pallas_call(
        _precompute_body,
        grid=(T, N // BN),
        in_specs=[
            pl.BlockSpec((BN, D), lambda t, i: (i, 0)),
            pl.BlockSpec((1, D, 2 * D), lambda t, i: (t, 0, 0)),
        ],
        out_specs=pl.BlockSpec((2, 1, BN, D), lambda t, i: (0, t, i, 0)),
        out_shape=jax.ShapeDtypeStruct((2, T, N, D), jnp.float32),
    )(ns, w0cat)


def _gather_rows(table, gidx):
    G = gidx.shape[0]
    per_w = G // NW
    C = 200
    steps = per_w // C
    mesh = plsc.VectorSubcoreMesh(core_axis_name="c", subcore_axis_name="s")

    @functools.partial(
        pl.kernel,
        out_type=jax.ShapeDtypeStruct((G, D), jnp.float32),
        mesh=mesh,
        scratch_types=[
            pltpu.VMEM((C,), jnp.int32),
            pltpu.VMEM((C, D), jnp.float32),
            pltpu.SemaphoreType.DMA,
        ],
    )
    def k(table_hbm, idx_hbm, out_hbm, idx_v, rows_v, sem):
        wid = lax.axis_index("s") * NC + lax.axis_index("c")
        base = wid * per_w

        @pl.loop(0, steps)
        def _(i):
            off = base + i * C
            pltpu.sync_copy(idx_hbm.at[pl.ds(off, C)], idx_v)
            pltpu.async_copy(table_hbm.at[idx_v], rows_v, sem).wait()
            pltpu.sync_copy(rows_v, out_hbm.at[pl.ds(off, C)])

    return k(table, gidx)


def _mlp_body(a_ref, b_ref, w_ref, m_ref):
    x = jnp.maximum(a_ref[0] + b_ref[0], 0.0)
    m = jnp.dot(x, w_ref[0], precision=HIGH, preferred_element_type=jnp.float32)
    m_ref[...] = jnp.maximum(m, 0.0)


def _mlp(g2, w1s):
    BE = 1000
    per_t = E // BE
    return pl.pallas_call(
        _mlp_body,
        grid=(T, per_t),
        in_specs=[
            pl.BlockSpec((1, BE, D), lambda t, i: (0, t * per_t + i, 0)),
            pl.BlockSpec((1, BE, D), lambda t, i: (1, t * per_t + i, 0)),
            pl.BlockSpec((1, D, D), lambda t, i: (t, 0, 0)),
        ],
        out_specs=pl.BlockSpec((BE, D), lambda t, i: (t * per_t + i, 0)),
        out_shape=jax.ShapeDtypeStruct((T * E, D), jnp.float32),
    )(g2, g2, w1s)


def _scatter_add(msgs, tgts, zeros):
    M = msgs.shape[0]
    per_w = M // NW
    C = 200
    steps = per_w // C
    rows_per_tile = 624
    tail = N - NS * rows_per_tile
    mesh = plsc.VectorSubcoreMesh(core_axis_name="c", subcore_axis_name="s")

    @functools.partial(
        pl.kernel,
        out_type=jax.ShapeDtypeStruct((NC, N, D), jnp.float32),
        mesh=mesh,
        scratch_types=[
            pltpu.VMEM((C,), jnp.int32),
            pltpu.VMEM((C, D), jnp.float32),
            pltpu.VMEM_SHARED((N, D), jnp.float32),
            pltpu.SemaphoreType.DMA,
        ],
    )
    def k(msg_hbm, tgt_hbm, zero_hbm, out_hbm, idx_v, m_v, acc, sem):
        cid = lax.axis_index("c")
        sid = lax.axis_index("s")
        wid = sid * NC + cid
        base = wid * per_w

        @pl.when(sid == 0)
        def _():
            pltpu.sync_copy(zero_hbm, acc)

        plsc.subcore_barrier()

        @pl.loop(0, steps)
        def _(i):
            off = base + i * C
            pltpu.sync_copy(tgt_hbm.at[pl.ds(off, C)], idx_v)
            pltpu.async_copy(msg_hbm.at[pl.ds(off, C)], m_v, sem).wait()
            pltpu.sync_copy(m_v, acc.at[idx_v], add=True)

        plsc.subcore_barrier()
        r0 = sid * rows_per_tile
        pltpu.sync_copy(acc.at[pl.ds(r0, rows_per_tile)],
                        out_hbm.at[cid, pl.ds(r0, rows_per_tile)])

        @pl.when(sid == 0)
        def _():
            t0 = NS * rows_per_tile
            pltpu.sync_copy(acc.at[pl.ds(t0, tail)],
                            out_hbm.at[cid, pl.ds(t0, tail)])

    return k(msgs, tgts, zeros)


def _sum_partials_body(p_ref, o_ref):
    o_ref[...] = p_ref[0] + p_ref[1]


def _sum_partials(partials):
    BN = 1000
    return pl.pallas_call(
        _sum_partials_body,
        grid=(N // BN,),
        in_specs=[pl.BlockSpec((2, BN, D), lambda i: (0, i, 0))],
        out_specs=pl.BlockSpec((BN, D), lambda i: (i, 0)),
        out_shape=jax.ShapeDtypeStruct((N, D), jnp.float32),
    )(partials)


def kernel(node_states, adj_list_0, adj_list_1, adj_list_2, adj_list_3,
           W_0_0, W_0_1, W_1_0, W_1_1, W_2_0, W_2_1, W_3_0, W_3_1):
    adj = [adj_list_0, adj_list_1, adj_list_2, adj_list_3]
    w0cat = (jnp.stack([W_0_0, W_1_0, W_2_0, W_3_0])
             .reshape(T, 2, D, D).transpose(0, 2, 1, 3).reshape(T, D, 2 * D))
    w1s = jnp.stack([W_0_1, W_1_1, W_2_1, W_3_1])

    srcs = jnp.concatenate([adj[t][:, 0] + t * N for t in range(T)])
    tgts = jnp.concatenate([adj[t][:, 1] for t in range(T)])
    gidx = jnp.concatenate([srcs, tgts + T * N + jnp.repeat(
        jnp.arange(T, dtype=jnp.int32) * N, E)])

    ab = _precompute(node_states, w0cat)
    table = ab.reshape(2 * T * N, D)
    g = _gather_rows(table, gidx)
    msgs = _mlp(g.reshape(2, T * E, D), w1s)
    zeros = jnp.zeros((N, D), jnp.float32)
    partials = _scatter_add(msgs, tgts, zeros)
    return _sum_partials(partials)


# speedup vs baseline: 3.8861x; 1.3656x over previous
import functools

import jax
import jax.numpy as jnp
from jax import lax
from jax.experimental import pallas as pl
from jax.experimental.pallas import tpu as pltpu
from jax.experimental.pallas import tpu_sc as plsc

N = 10000
D = 128
E = 80000
T = 4

NC = 2
NS = 16
NW = NC * NS

PREC = jax.lax.Precision.DEFAULT


def _precompute_body(ns_ref, w_ref, ab_ref):
    x = ns_ref[...]
    w = w_ref[0]
    y = jnp.dot(x, w, precision=PREC, preferred_element_type=jnp.float32)
    ab_ref[0, 0] = y[:, :D]
    ab_ref[1, 0] = y[:, D:]


def _precompute(ns, w0cat):
    BN = 2000
    return pl.pallas_call(
        _precompute_body,
        grid=(T, N // BN),
        in_specs=[
            pl.BlockSpec((BN, D), lambda t, i: (i, 0)),
            pl.BlockSpec((1, D, 2 * D), lambda t, i: (t, 0, 0)),
        ],
        out_specs=pl.BlockSpec((2, 1, BN, D), lambda t, i: (0, t, i, 0)),
        out_shape=jax.ShapeDtypeStruct((2, T, N, D), jnp.float32),
    )(ns, w0cat)


def _gather_rows(table, gidx):
    G = gidx.shape[0]
    per_w = G // NW
    C = 200
    steps = per_w // C
    assert steps >= 12
    steady_end = 8 + ((steps - 12) // 4) * 4
    mesh = plsc.VectorSubcoreMesh(core_axis_name="c", subcore_axis_name="s")

    @functools.partial(
        pl.kernel,
        out_type=jax.ShapeDtypeStruct((G, D), jnp.float32),
        mesh=mesh,
        scratch_types=(
            [pltpu.VMEM((C,), jnp.int32) for _ in range(4)]
            + [pltpu.VMEM((C, D), jnp.float32) for _ in range(4)]
            + [pltpu.SemaphoreType.DMA] * 12
        ),
    )
    def k(table_hbm, idx_hbm, out_hbm, *scratch):
        ib = scratch[0:4]
        rb = scratch[4:8]
        si = scratch[8:12]
        sg = scratch[12:16]
        so = scratch[16:20]
        wid = lax.axis_index("s") * NC + lax.axis_index("c")
        base = wid * per_w

        def idx_cp(kk, r):
            return pltpu.make_async_copy(
                idx_hbm.at[pl.ds(base + kk * C, C)], ib[r], si[r])

        def gat(kk, r):
            del kk
            return pltpu.make_async_copy(table_hbm.at[ib[r]], rb[r], sg[r])

        def out_cp(kk, r):
            return pltpu.make_async_copy(
                rb[r], out_hbm.at[pl.ds(base + kk * C, C)], so[r])

        def ops(kk, r, first=False, last=False):
            rp = (r - 1) % 4
            if not first:
                gat(kk - 1, rp).wait()
                out_cp(kk - 1, rp).start()
            if not last:
                idx_cp(kk + 3, rp).start()
            if not isinstance(kk, int) or kk >= 4:
                out_cp(kk - 4, r).wait()
            idx_cp(kk, r).wait()
            gat(kk, r).start()

        for kk in range(3):
            idx_cp(kk, kk % 4).start()
        for kk in range(8):
            ops(kk, kk % 4, first=(kk == 0), last=(kk + 3 >= steps))

        @pl.loop(8, steady_end, step=4)
        def _(k0):
            for j in range(4):
                ops(k0 + j, j)

        for kk in range(steady_end, steps):
            ops(kk, kk % 4, last=(kk + 3 >= steps))

        gat(steps - 1, (steps - 1) % 4).wait()
        out_cp(steps - 1, (steps - 1) % 4).start()
        for kk in range(steps - 4, steps):
            out_cp(kk, kk % 4).wait()

    return k(table, gidx)


def _mlp_body(a_ref, b_ref, w_ref, m_ref):
    x = jnp.maximum(a_ref[0] + b_ref[0], 0.0)
    m = jnp.dot(x, w_ref[0], precision=PREC, preferred_element_type=jnp.float32)
    m_ref[...] = jnp.maximum(m, 0.0)


def _mlp(g2, w1s):
    BE = 2000
    per_t = E // BE
    return pl.pallas_call(
        _mlp_body,
        grid=(T, per_t),
        in_specs=[
            pl.BlockSpec((1, BE, D), lambda t, i: (0, t * per_t + i, 0)),
            pl.BlockSpec((1, BE, D), lambda t, i: (1, t * per_t + i, 0)),
            pl.BlockSpec((1, D, D), lambda t, i: (t, 0, 0)),
        ],
        out_specs=pl.BlockSpec((BE, D), lambda t, i: (t * per_t + i, 0)),
        out_shape=jax.ShapeDtypeStruct((T * E, D), jnp.float32),
    )(g2, g2, w1s)


def _scatter_add(msgs, tgts, zeros):
    M = msgs.shape[0]
    per_w = M // NW
    C = 200
    steps = per_w // C
    rows_per_tile = 624
    tail = N - NS * rows_per_tile
    mesh = plsc.VectorSubcoreMesh(core_axis_name="c", subcore_axis_name="s")

    @functools.partial(
        pl.kernel,
        out_type=jax.ShapeDtypeStruct((NC, N, D), jnp.float32),
        mesh=mesh,
        scratch_types=(
            [pltpu.VMEM((C,), jnp.int32) for _ in range(2)]
            + [pltpu.VMEM((C, D), jnp.float32)]
            + [pltpu.VMEM_SHARED((N, D), jnp.float32)]
            + [pltpu.SemaphoreType.DMA] * 3
        ),
    )
    def k(msg_hbm, tgt_hbm, zero_hbm, out_hbm, ib0, ib1, mb, acc, si0, si1, sm):
        ib = (ib0, ib1)
        si = (si0, si1)
        cid = lax.axis_index("c")
        sid = lax.axis_index("s")
        wid = sid * NC + cid
        base = wid * per_w

        @pl.when(sid == 0)
        def _():
            pltpu.sync_copy(zero_hbm, acc)

        plsc.subcore_barrier()

        def idx_cp(kk, r):
            return pltpu.make_async_copy(
                tgt_hbm.at[pl.ds(base + kk * C, C)], ib[r], si[r])

        def msg_cp(kk):
            return pltpu.make_async_copy(
                msg_hbm.at[pl.ds(base + kk * C, C)], mb, sm)

        idx_cp(0, 0).start()
        msg_cp(0).start()

        @pl.loop(0, steps, step=2)
        def _(k0):
            for j in range(2):
                kk = k0 + j
                r = j
                idx_cp(kk, r).wait()
                msg_cp(kk).wait()
                pl.when(kk + 1 < steps)(lambda: idx_cp(kk + 1, 1 - r).start())
                pltpu.sync_copy(mb, acc.at[ib[r]], add=True)
                pl.when(kk + 1 < steps)(lambda: msg_cp(kk + 1).start())

        plsc.subcore_barrier()
        r0 = sid * rows_per_tile
        pltpu.sync_copy(acc.at[pl.ds(r0, rows_per_tile)],
                        out_hbm.at[cid, pl.ds(r0, rows_per_tile)])

        @pl.when(sid == 0)
        def _():
            t0 = NS * rows_per_tile
            pltpu.sync_copy(acc.at[pl.ds(t0, tail)],
                            out_hbm.at[cid, pl.ds(t0, tail)])

    return k(msgs, tgts, zeros)


def _sum_partials_body(p_ref, o_ref):
    o_ref[...] = p_ref[0] + p_ref[1]


def _sum_partials(partials):
    BN = 2000
    return pl.pallas_call(
        _sum_partials_body,
        grid=(N // BN,),
        in_specs=[pl.BlockSpec((2, BN, D), lambda i: (0, i, 0))],
        out_specs=pl.BlockSpec((BN, D), lambda i: (i, 0)),
        out_shape=jax.ShapeDtypeStruct((N, D), jnp.float32),
    )(partials)


def kernel(node_states, adj_list_0, adj_list_1, adj_list_2, adj_list_3,
           W_0_0, W_0_1, W_1_0, W_1_1, W_2_0, W_2_1, W_3_0, W_3_1):
    adj = [adj_list_0, adj_list_1, adj_list_2, adj_list_3]
    w0cat = (jnp.stack([W_0_0, W_1_0, W_2_0, W_3_0])
             .reshape(T, 2, D, D).transpose(0, 2, 1, 3).reshape(T, D, 2 * D))
    w1s = jnp.stack([W_0_1, W_1_1, W_2_1, W_3_1])

    srcs = jnp.concatenate([adj[t][:, 0] + t * N for t in range(T)])
    tgts = jnp.concatenate([adj[t][:, 1] for t in range(T)])
    gidx = jnp.concatenate([srcs, tgts + T * N + jnp.repeat(
        jnp.arange(T, dtype=jnp.int32) * N, E)])

    ab = _precompute(node_states, w0cat)
    table = ab.reshape(2 * T * N, D)
    g = _gather_rows(table, gidx)
    msgs = _mlp(g.reshape(2, T * E, D), w1s)
    zeros = jnp.zeros((N, D), jnp.float32)
    partials = _scatter_add(msgs, tgts, zeros)
    return _sum_partials(partials)


# speedup vs baseline: 4.4736x; 1.1512x over previous
import functools

import jax
import jax.numpy as jnp
from jax import lax
from jax.experimental import pallas as pl
from jax.experimental.pallas import tpu as pltpu
from jax.experimental.pallas import tpu_sc as plsc

N = 10000
D = 128
E = 80000
T = 4

NC = 2
NS = 16
NW = NC * NS

PREC = jax.lax.Precision.DEFAULT


def _precompute_body(ns_ref, w_ref, ab_ref):
    x = ns_ref[...]
    w = w_ref[0]
    y = jnp.dot(x, w, precision=PREC, preferred_element_type=jnp.float32)
    ab_ref[0, 0] = y[:, :D]
    ab_ref[1, 0] = y[:, D:]


def _precompute(ns, w0cat):
    BN = 2000
    return pl.pallas_call(
        _precompute_body,
        grid=(T, N // BN),
        in_specs=[
            pl.BlockSpec((BN, D), lambda t, i: (i, 0)),
            pl.BlockSpec((1, D, 2 * D), lambda t, i: (t, 0, 0)),
        ],
        out_specs=pl.BlockSpec((2, 1, BN, D), lambda t, i: (0, t, i, 0)),
        out_shape=jax.ShapeDtypeStruct((2, T, N, D), jnp.float32),
    )(ns, w0cat)


def _gather_rows(table, gidx):
    G = gidx.shape[0]
    per_w = G // NW
    C = 200
    steps = per_w // C
    assert steps >= 12
    steady_end = 8 + ((steps - 12) // 4) * 4
    mesh = plsc.VectorSubcoreMesh(core_axis_name="c", subcore_axis_name="s")

    @functools.partial(
        pl.kernel,
        out_type=jax.ShapeDtypeStruct((G, D), jnp.float32),
        mesh=mesh,
        scratch_types=(
            [pltpu.VMEM((C,), jnp.int32) for _ in range(4)]
            + [pltpu.VMEM((C, D), jnp.float32) for _ in range(4)]
            + [pltpu.SemaphoreType.DMA] * 12
        ),
    )
    def k(table_hbm, idx_hbm, out_hbm, *scratch):
        ib = scratch[0:4]
        rb = scratch[4:8]
        si = scratch[8:12]
        sg = scratch[12:16]
        so = scratch[16:20]
        wid = lax.axis_index("s") * NC + lax.axis_index("c")
        base = wid * per_w

        def idx_cp(kk, r):
            return pltpu.make_async_copy(
                idx_hbm.at[pl.ds(base + kk * C, C)], ib[r], si[r])

        def gat(kk, r):
            del kk
            return pltpu.make_async_copy(table_hbm.at[ib[r]], rb[r], sg[r])

        def out_cp(kk, r):
            return pltpu.make_async_copy(
                rb[r], out_hbm.at[pl.ds(base + kk * C, C)], so[r])

        def ops(kk, r, first=False, last=False):
            rp = (r - 1) % 4
            if not first:
                gat(kk - 1, rp).wait()
                out_cp(kk - 1, rp).start()
            if not last:
                idx_cp(kk + 3, rp).start()
            if not isinstance(kk, int) or kk >= 4:
                out_cp(kk - 4, r).wait()
            idx_cp(kk, r).wait()
            gat(kk, r).start()

        for kk in range(3):
            idx_cp(kk, kk % 4).start()
        for kk in range(8):
            ops(kk, kk % 4, first=(kk == 0), last=(kk + 3 >= steps))

        @pl.loop(8, steady_end, step=4)
        def _(k0):
            for j in range(4):
                ops(k0 + j, j)

        for kk in range(steady_end, steps):
            ops(kk, kk % 4, last=(kk + 3 >= steps))

        gat(steps - 1, (steps - 1) % 4).wait()
        out_cp(steps - 1, (steps - 1) % 4).start()
        for kk in range(steps - 4, steps):
            out_cp(kk, kk % 4).wait()

    return k(table, gidx)


def _mlp_body(a_ref, b_ref, w_ref, m_ref):
    x = jnp.maximum(a_ref[0] + b_ref[0], 0.0)
    m = jnp.dot(x, w_ref[0], precision=PREC, preferred_element_type=jnp.float32)
    m_ref[...] = jnp.maximum(m, 0.0)


def _mlp(g2, w1s):
    TT = w1s.shape[0]
    BE = 2000
    per_t = E // BE
    return pl.pallas_call(
        _mlp_body,
        grid=(TT, per_t),
        in_specs=[
            pl.BlockSpec((1, BE, D), lambda t, i: (0, t * per_t + i, 0)),
            pl.BlockSpec((1, BE, D), lambda t, i: (1, t * per_t + i, 0)),
            pl.BlockSpec((1, D, D), lambda t, i: (t, 0, 0)),
        ],
        out_specs=pl.BlockSpec((BE, D), lambda t, i: (t * per_t + i, 0)),
        out_shape=jax.ShapeDtypeStruct((TT * E, D), jnp.float32),
    )(g2, g2, w1s)


def _scatter_add(msgs, tgts, init):
    M = msgs.shape[0]
    per_w = M // NW
    C = 200
    steps = per_w // C
    rows_per_tile = 624
    tail = N - NS * rows_per_tile
    mesh = plsc.VectorSubcoreMesh(core_axis_name="c", subcore_axis_name="s")

    @functools.partial(
        pl.kernel,
        out_type=jax.ShapeDtypeStruct((NC, N, D), jnp.float32),
        mesh=mesh,
        scratch_types=(
            [pltpu.VMEM((C,), jnp.int32) for _ in range(2)]
            + [pltpu.VMEM((C, D), jnp.float32)]
            + [pltpu.VMEM_SHARED((N, D), jnp.float32)]
            + [pltpu.SemaphoreType.DMA] * 3
        ),
    )
    def k(msg_hbm, tgt_hbm, init_hbm, out_hbm, ib0, ib1, mb, acc, si0, si1, sm):
        ib = (ib0, ib1)
        si = (si0, si1)
        cid = lax.axis_index("c")
        sid = lax.axis_index("s")
        wid = sid * NC + cid
        base = wid * per_w

        @pl.when(sid == 0)
        def _():
            pltpu.sync_copy(init_hbm.at[cid], acc)

        plsc.subcore_barrier()

        def idx_cp(kk, r):
            return pltpu.make_async_copy(
                tgt_hbm.at[pl.ds(base + kk * C, C)], ib[r], si[r])

        def msg_cp(kk):
            return pltpu.make_async_copy(
                msg_hbm.at[pl.ds(base + kk * C, C)], mb, sm)

        idx_cp(0, 0).start()
        msg_cp(0).start()
        even = steps // 2 * 2

        @pl.loop(0, even, step=2)
        def _(k0):
            for j in range(2):
                kk = k0 + j
                r = j
                idx_cp(kk, r).wait()
                msg_cp(kk).wait()
                pl.when(kk + 1 < steps)(lambda: idx_cp(kk + 1, 1 - r).start())
                pltpu.sync_copy(mb, acc.at[ib[r]], add=True)
                pl.when(kk + 1 < steps)(lambda: msg_cp(kk + 1).start())

        if steps % 2:
            kk = steps - 1
            idx_cp(kk, kk % 2).wait()
            msg_cp(kk).wait()
            pltpu.sync_copy(mb, acc.at[ib[kk % 2]], add=True)

        plsc.subcore_barrier()
        r0 = sid * rows_per_tile
        pltpu.sync_copy(acc.at[pl.ds(r0, rows_per_tile)],
                        out_hbm.at[cid, pl.ds(r0, rows_per_tile)])

        @pl.when(sid == 0)
        def _():
            t0 = NS * rows_per_tile
            pltpu.sync_copy(acc.at[pl.ds(t0, tail)],
                            out_hbm.at[cid, pl.ds(t0, tail)])

    return k(msgs, tgts, init)


def _sum_partials_body(p_ref, o_ref):
    o_ref[...] = p_ref[0] + p_ref[1]


def _sum_partials(partials):
    BN = 2000
    return pl.pallas_call(
        _sum_partials_body,
        grid=(N // BN,),
        in_specs=[pl.BlockSpec((2, BN, D), lambda i: (0, i, 0))],
        out_specs=pl.BlockSpec((BN, D), lambda i: (i, 0)),
        out_shape=jax.ShapeDtypeStruct((N, D), jnp.float32),
    )(partials)


def kernel(node_states, adj_list_0, adj_list_1, adj_list_2, adj_list_3,
           W_0_0, W_0_1, W_1_0, W_1_1, W_2_0, W_2_1, W_3_0, W_3_1):
    adj = [adj_list_0, adj_list_1, adj_list_2, adj_list_3]
    w0cat = (jnp.stack([W_0_0, W_1_0, W_2_0, W_3_0])
             .reshape(T, 2, D, D).transpose(0, 2, 1, 3).reshape(T, D, 2 * D))
    w1s = jnp.stack([W_0_1, W_1_1, W_2_1, W_3_1])

    ab = _precompute(node_states, w0cat)
    table = ab.reshape(2 * T * N, D)

    partials = jnp.zeros((NC, N, D), jnp.float32)
    for types in ((0, 1), (2, 3)):
        gidx = jnp.concatenate(
            [adj[t][:, 0] + t * N for t in types]
            + [adj[t][:, 1] + (T + t) * N for t in types])
        tgts = jnp.concatenate([adj[t][:, 1] for t in types])
        g = _gather_rows(table, gidx)
        msgs = _mlp(g.reshape(2, len(types) * E, D),
                    w1s[types[0]:types[-1] + 1])
        partials = _scatter_add(msgs, tgts, partials)
    return _sum_partials(partials)


# speedup vs baseline: 5.3495x; 1.1958x over previous
import functools

import jax
import jax.numpy as jnp
from jax import lax
from jax.experimental import pallas as pl
from jax.experimental.pallas import tpu as pltpu
from jax.experimental.pallas import tpu_sc as plsc

N = 10000
D = 128
E = 80000
T = 4

NC = 2
NS = 16
NW = NC * NS

PREC = jax.lax.Precision.DEFAULT


def _precompute_body(ns_ref, w_ref, ab_ref):
    x = ns_ref[...]
    w = w_ref[0]
    y = jnp.dot(x, w, precision=PREC, preferred_element_type=jnp.float32)
    ab_ref[0, 0] = y[:, :D]
    ab_ref[1, 0] = y[:, D:]


def _precompute(ns, w0cat):
    BN = 2000
    return pl.pallas_call(
        _precompute_body,
        grid=(T, N // BN),
        in_specs=[
            pl.BlockSpec((BN, D), lambda t, i: (i, 0)),
            pl.BlockSpec((1, D, 2 * D), lambda t, i: (t, 0, 0)),
        ],
        out_specs=pl.BlockSpec((2, 1, BN, D), lambda t, i: (0, t, i, 0)),
        out_shape=jax.ShapeDtypeStruct((2, T, N, D), jnp.float32),
    )(ns, w0cat)


def _gather_add(table, sidx, tidx):
    M = sidx.shape[0]
    per_w = M // NW
    C = 200
    steps = per_w // C
    assert steps >= 4 and per_w % C == 0
    mesh = plsc.VectorSubcoreMesh(core_axis_name="c", subcore_axis_name="s")

    @functools.partial(
        pl.kernel,
        out_type=jax.ShapeDtypeStruct((M, D), jnp.float32),
        mesh=mesh,
        scratch_types=(
            [pltpu.VMEM((C,), jnp.int32) for _ in range(4)]
            + [pltpu.VMEM((C, D), jnp.float32) for _ in range(4)]
            + [pltpu.SemaphoreType.DMA] * 10
        ),
    )
    def k(table_hbm, sidx_hbm, tidx_hbm, out_hbm, *scratch):
        isb = scratch[0:2]
        itb = scratch[2:4]
        ra = scratch[4:6]
        rb = scratch[6:8]
        ss = scratch[8:10]
        st = scratch[10:12]
        sa = scratch[12:14]
        sb = scratch[14:16]
        so = scratch[16:18]
        wid = lax.axis_index("s") * NC + lax.axis_index("c")
        base = wid * per_w

        def is_cp(kk, r):
            return pltpu.make_async_copy(
                sidx_hbm.at[pl.ds(base + kk * C, C)], isb[r], ss[r])

        def it_cp(kk, r):
            return pltpu.make_async_copy(
                tidx_hbm.at[pl.ds(base + kk * C, C)], itb[r], st[r])

        def gat_a(r):
            return pltpu.make_async_copy(table_hbm.at[isb[r]], ra[r], sa[r])

        def gat_b(r):
            return pltpu.make_async_copy(table_hbm.at[itb[r]], rb[r], sb[r])

        def add_rows(r):
            @plsc.parallel_loop(0, C, unroll=2)
            def _(i):
                for c in range(0, D, 16):
                    ra[r][i, pl.ds(c, 16)] += rb[r][i, pl.ds(c, 16)]

        def out_cp(kk, r):
            return pltpu.make_async_copy(
                ra[r], out_hbm.at[pl.ds(base + kk * C, C)], so[r])

        def ops(kk, r, first=False, second=False, last=False):
            rp = 1 - r
            if not (first or second):
                out_cp(kk - 2, r).wait()
            is_cp(kk, r).wait()
            it_cp(kk, r).wait()
            gat_a(r).start()
            gat_b(r).start()
            if not first:
                gat_a(rp).wait()
                gat_b(rp).wait()
                add_rows(rp)
                out_cp(kk - 1, rp).start()
                if not last:
                    is_cp(kk + 1, rp).start()
                    it_cp(kk + 1, rp).start()

        is_cp(0, 0).start()
        it_cp(0, 0).start()
        is_cp(1, 1).start()
        it_cp(1, 1).start()
        ops(0, 0, first=True)
        ops(1, 1, second=True, last=(steps == 2))

        even_lo = 2
        even_hi = even_lo + max(0, steps - even_lo - 2) // 2 * 2

        @pl.loop(even_lo, even_hi, step=2)
        def _(k0):
            for j in range(2):
                ops(k0 + j, j)

        for kk in range(even_hi, steps):
            ops(kk, kk % 2, last=(kk + 1 >= steps))

        r = (steps - 1) % 2
        gat_a(r).wait()
        gat_b(r).wait()
        add_rows(r)
        out_cp(steps - 1, r).start()
        out_cp(steps - 2, 1 - r).wait()
        out_cp(steps - 1, r).wait()

    return k(table, sidx, tidx)


def _mlp_body(h_ref, w_ref, m_ref):
    x = jnp.maximum(h_ref[...], 0.0)
    m = jnp.dot(x, w_ref[0], precision=PREC, preferred_element_type=jnp.float32)
    m_ref[...] = jnp.maximum(m, 0.0)


def _mlp(h, w1s):
    TT = w1s.shape[0]
    BE = 2000
    per_t = E // BE
    return pl.pallas_call(
        _mlp_body,
        grid=(TT, per_t),
        in_specs=[
            pl.BlockSpec((BE, D), lambda t, i: (t * per_t + i, 0)),
            pl.BlockSpec((1, D, D), lambda t, i: (t, 0, 0)),
        ],
        out_specs=pl.BlockSpec((BE, D), lambda t, i: (t * per_t + i, 0)),
        out_shape=jax.ShapeDtypeStruct((TT * E, D), jnp.float32),
    )(h, w1s)


def _scatter_add(msgs, tgts, init):
    M = msgs.shape[0]
    per_w = M // NW
    C = 200
    steps = per_w // C
    rows_per_tile = 624
    tail = N - NS * rows_per_tile
    mesh = plsc.VectorSubcoreMesh(core_axis_name="c", subcore_axis_name="s")

    @functools.partial(
        pl.kernel,
        out_type=jax.ShapeDtypeStruct((NC, N, D), jnp.float32),
        mesh=mesh,
        scratch_types=(
            [pltpu.VMEM((C,), jnp.int32) for _ in range(2)]
            + [pltpu.VMEM((C, D), jnp.float32)]
            + [pltpu.VMEM_SHARED((N, D), jnp.float32)]
            + [pltpu.SemaphoreType.DMA] * 3
        ),
    )
    def k(msg_hbm, tgt_hbm, init_hbm, out_hbm, ib0, ib1, mb, acc, si0, si1, sm):
        ib = (ib0, ib1)
        si = (si0, si1)
        cid = lax.axis_index("c")
        sid = lax.axis_index("s")
        wid = sid * NC + cid
        base = wid * per_w

        @pl.when(sid == 0)
        def _():
            pltpu.sync_copy(init_hbm.at[cid], acc)

        plsc.subcore_barrier()

        def idx_cp(kk, r):
            return pltpu.make_async_copy(
                tgt_hbm.at[pl.ds(base + kk * C, C)], ib[r], si[r])

        def msg_cp(kk):
            return pltpu.make_async_copy(
                msg_hbm.at[pl.ds(base + kk * C, C)], mb, sm)

        idx_cp(0, 0).start()
        msg_cp(0).start()
        even = steps // 2 * 2

        @pl.loop(0, even, step=2)
        def _(k0):
            for j in range(2):
                kk = k0 + j
                r = j
                idx_cp(kk, r).wait()
                msg_cp(kk).wait()
                pl.when(kk + 1 < steps)(lambda: idx_cp(kk + 1, 1 - r).start())
                pltpu.sync_copy(mb, acc.at[ib[r]], add=True)
                pl.when(kk + 1 < steps)(lambda: msg_cp(kk + 1).start())

        if steps % 2:
            kk = steps - 1
            idx_cp(kk, kk % 2).wait()
            msg_cp(kk).wait()
            pltpu.sync_copy(mb, acc.at[ib[kk % 2]], add=True)

        plsc.subcore_barrier()
        r0 = sid * rows_per_tile
        pltpu.sync_copy(acc.at[pl.ds(r0, rows_per_tile)],
                        out_hbm.at[cid, pl.ds(r0, rows_per_tile)])

        @pl.when(sid == 0)
        def _():
            t0 = NS * rows_per_tile
            pltpu.sync_copy(acc.at[pl.ds(t0, tail)],
                            out_hbm.at[cid, pl.ds(t0, tail)])

    return k(msgs, tgts, init)


def _sum_partials_body(p_ref, o_ref):
    o_ref[...] = p_ref[0] + p_ref[1]


def _sum_partials(partials):
    BN = 2000
    return pl.pallas_call(
        _sum_partials_body,
        grid=(N // BN,),
        in_specs=[pl.BlockSpec((2, BN, D), lambda i: (0, i, 0))],
        out_specs=pl.BlockSpec((BN, D), lambda i: (i, 0)),
        out_shape=jax.ShapeDtypeStruct((N, D), jnp.float32),
    )(partials)


def kernel(node_states, adj_list_0, adj_list_1, adj_list_2, adj_list_3,
           W_0_0, W_0_1, W_1_0, W_1_1, W_2_0, W_2_1, W_3_0, W_3_1):
    adj = [adj_list_0, adj_list_1, adj_list_2, adj_list_3]
    w0cat = (jnp.stack([W_0_0, W_1_0, W_2_0, W_3_0])
             .reshape(T, 2, D, D).transpose(0, 2, 1, 3).reshape(T, D, 2 * D))
    w1s = jnp.stack([W_0_1, W_1_1, W_2_1, W_3_1])

    ab = _precompute(node_states, w0cat)
    table = ab.reshape(2 * T * N, D)

    partials = jnp.zeros((NC, N, D), jnp.float32)
    for types in ((0, 1), (2, 3)):
        sidx = jnp.concatenate([adj[t][:, 0] + t * N for t in types])
        tidx = jnp.concatenate([adj[t][:, 1] + (T + t) * N for t in types])
        tgts = jnp.concatenate([adj[t][:, 1] for t in types])
        h = _gather_add(table, sidx, tidx)
        msgs = _mlp(h, w1s[types[0]:types[-1] + 1])
        partials = _scatter_add(msgs, tgts, partials)
    return _sum_partials(partials)


# speedup vs baseline: 5.4015x; 1.0097x over previous
import functools

import jax
import jax.numpy as jnp
from jax import lax
from jax.experimental import pallas as pl
from jax.experimental.pallas import tpu as pltpu
from jax.experimental.pallas import tpu_sc as plsc

N = 10000
D = 128
E = 80000
T = 4

NC = 2
NS = 16
NW = NC * NS

PREC = jax.lax.Precision.DEFAULT


def _precompute_body(ns_ref, w_ref, ab_ref):
    x = ns_ref[...]
    w = w_ref[0]
    y = jnp.dot(x, w, precision=PREC, preferred_element_type=jnp.float32)
    ab_ref[0, 0] = y[:, :D]
    ab_ref[1, 0] = y[:, D:]


def _precompute(ns, w0cat):
    TT = w0cat.shape[0]
    BN = 2000
    return pl.pallas_call(
        _precompute_body,
        grid=(TT, N // BN),
        in_specs=[
            pl.BlockSpec((BN, D), lambda t, i: (i, 0)),
            pl.BlockSpec((1, D, 2 * D), lambda t, i: (t, 0, 0)),
        ],
        out_specs=pl.BlockSpec((2, 1, BN, D), lambda t, i: (0, t, i, 0)),
        out_shape=jax.ShapeDtypeStruct((2, TT, N, D), jnp.float32),
    )(ns, w0cat)


def _gather_add(table, sidx, tidx):
    M = sidx.shape[0]
    per_w = M // NW
    C = 200
    steps = per_w // C
    assert steps >= 4 and per_w % C == 0
    mesh = plsc.VectorSubcoreMesh(core_axis_name="c", subcore_axis_name="s")

    @functools.partial(
        pl.kernel,
        out_type=jax.ShapeDtypeStruct((M, D), jnp.float32),
        mesh=mesh,
        scratch_types=(
            [pltpu.VMEM((C,), jnp.int32) for _ in range(4)]
            + [pltpu.VMEM((C, D), jnp.float32) for _ in range(4)]
            + [pltpu.SemaphoreType.DMA] * 10
        ),
    )
    def k(table_hbm, sidx_hbm, tidx_hbm, out_hbm, *scratch):
        isb = scratch[0:2]
        itb = scratch[2:4]
        ra = scratch[4:6]
        rb = scratch[6:8]
        ss = scratch[8:10]
        st = scratch[10:12]
        sa = scratch[12:14]
        sb = scratch[14:16]
        so = scratch[16:18]
        wid = lax.axis_index("s") * NC + lax.axis_index("c")
        base = wid * per_w

        def is_cp(kk, r):
            return pltpu.make_async_copy(
                sidx_hbm.at[pl.ds(base + kk * C, C)], isb[r], ss[r])

        def it_cp(kk, r):
            return pltpu.make_async_copy(
                tidx_hbm.at[pl.ds(base + kk * C, C)], itb[r], st[r])

        def gat_a(r):
            return pltpu.make_async_copy(table_hbm.at[isb[r]], ra[r], sa[r])

        def gat_b(r):
            return pltpu.make_async_copy(table_hbm.at[itb[r]], rb[r], sb[r])

        def add_rows(r):
            @plsc.parallel_loop(0, C, unroll=2)
            def _(i):
                for c in range(0, D, 16):
                    ra[r][i, pl.ds(c, 16)] += rb[r][i, pl.ds(c, 16)]

        def out_cp(kk, r):
            return pltpu.make_async_copy(
                ra[r], out_hbm.at[pl.ds(base + kk * C, C)], so[r])

        def ops(kk, r, first=False, second=False, last=False):
            rp = 1 - r
            if not (first or second):
                out_cp(kk - 2, r).wait()
            is_cp(kk, r).wait()
            it_cp(kk, r).wait()
            gat_a(r).start()
            gat_b(r).start()
            if not first:
                gat_a(rp).wait()
                gat_b(rp).wait()
                add_rows(rp)
                out_cp(kk - 1, rp).start()
                if not last:
                    is_cp(kk + 1, rp).start()
                    it_cp(kk + 1, rp).start()

        is_cp(0, 0).start()
        it_cp(0, 0).start()
        is_cp(1, 1).start()
        it_cp(1, 1).start()
        ops(0, 0, first=True)
        ops(1, 1, second=True, last=(steps == 2))

        even_lo = 2
        even_hi = even_lo + max(0, steps - even_lo - 2) // 2 * 2

        @pl.loop(even_lo, even_hi, step=2)
        def _(k0):
            for j in range(2):
                ops(k0 + j, j)

        for kk in range(even_hi, steps):
            ops(kk, kk % 2, last=(kk + 1 >= steps))

        r = (steps - 1) % 2
        gat_a(r).wait()
        gat_b(r).wait()
        add_rows(r)
        out_cp(steps - 1, r).start()
        out_cp(steps - 2, 1 - r).wait()
        out_cp(steps - 1, r).wait()

    return k(table, sidx, tidx)


def _mlp_body(h_ref, w_ref, m_ref):
    x = jnp.maximum(h_ref[...], 0.0)
    m = jnp.dot(x, w_ref[0], precision=PREC, preferred_element_type=jnp.float32)
    m_ref[...] = jnp.maximum(m, 0.0)


def _mlp(h, w1s):
    TT = w1s.shape[0]
    BE = 2000
    per_t = E // BE
    return pl.pallas_call(
        _mlp_body,
        grid=(TT, per_t),
        in_specs=[
            pl.BlockSpec((BE, D), lambda t, i: (t * per_t + i, 0)),
            pl.BlockSpec((1, D, D), lambda t, i: (t, 0, 0)),
        ],
        out_specs=pl.BlockSpec((BE, D), lambda t, i: (t * per_t + i, 0)),
        out_shape=jax.ShapeDtypeStruct((TT * E, D), jnp.float32),
    )(h, w1s)


def _scatter_add(msgs, tgts, init=None):
    M = msgs.shape[0]
    per_w = M // NW
    C = 200
    steps = per_w // C
    rows_per_tile = 624
    tail = N - NS * rows_per_tile
    mesh = plsc.VectorSubcoreMesh(core_axis_name="c", subcore_axis_name="s")

    scratch = (
        [pltpu.VMEM((C,), jnp.int32) for _ in range(2)]
        + [pltpu.VMEM((C, D), jnp.float32)]
        + [pltpu.VMEM_SHARED((N, D), jnp.float32)]
        + [pltpu.SemaphoreType.DMA] * 3
    )

    def body(msg_hbm, tgt_hbm, init_hbm, out_hbm,
             ib0, ib1, mb, acc, si0, si1, sm):
        ib = (ib0, ib1)
        si = (si0, si1)
        cid = lax.axis_index("c")
        sid = lax.axis_index("s")
        wid = sid * NC + cid
        base = wid * per_w

        if init_hbm is None:
            zvec = jnp.zeros((16,), jnp.float32)

            @plsc.parallel_loop(0, C)
            def _(i):
                for c in range(0, D, 16):
                    mb[i, pl.ds(c, 16)] = zvec

            r0z = sid * (N // NS)
            for j in range(3):
                pltpu.sync_copy(mb, acc.at[pl.ds(r0z + j * C, C)])
            pltpu.sync_copy(mb.at[pl.ds(0, N // NS - 3 * C)],
                            acc.at[pl.ds(r0z + 3 * C, N // NS - 3 * C)])
        else:
            @pl.when(sid == 0)
            def _():
                pltpu.sync_copy(init_hbm.at[cid], acc)

        plsc.subcore_barrier()

        def idx_cp(kk, r):
            return pltpu.make_async_copy(
                tgt_hbm.at[pl.ds(base + kk * C, C)], ib[r], si[r])

        def msg_cp(kk):
            return pltpu.make_async_copy(
                msg_hbm.at[pl.ds(base + kk * C, C)], mb, sm)

        idx_cp(0, 0).start()
        msg_cp(0).start()
        even = steps // 2 * 2

        @pl.loop(0, even, step=2)
        def _(k0):
            for j in range(2):
                kk = k0 + j
                r = j
                idx_cp(kk, r).wait()
                msg_cp(kk).wait()
                pl.when(kk + 1 < steps)(lambda: idx_cp(kk + 1, 1 - r).start())
                pltpu.sync_copy(mb, acc.at[ib[r]], add=True)
                pl.when(kk + 1 < steps)(lambda: msg_cp(kk + 1).start())

        if steps % 2:
            kk = steps - 1
            idx_cp(kk, kk % 2).wait()
            msg_cp(kk).wait()
            pltpu.sync_copy(mb, acc.at[ib[kk % 2]], add=True)

        plsc.subcore_barrier()
        r0 = sid * rows_per_tile
        pltpu.sync_copy(acc.at[pl.ds(r0, rows_per_tile)],
                        out_hbm.at[cid, pl.ds(r0, rows_per_tile)])

        @pl.when(sid == 0)
        def _():
            t0 = NS * rows_per_tile
            pltpu.sync_copy(acc.at[pl.ds(t0, tail)],
                            out_hbm.at[cid, pl.ds(t0, tail)])

    if init is None:
        @functools.partial(
            pl.kernel,
            out_type=jax.ShapeDtypeStruct((NC, N, D), jnp.float32),
            mesh=mesh, scratch_types=scratch)
        def k0(msg_hbm, tgt_hbm, out_hbm, *s):
            body(msg_hbm, tgt_hbm, None, out_hbm, *s)

        return k0(msgs, tgts)

    @functools.partial(
        pl.kernel,
        out_type=jax.ShapeDtypeStruct((NC, N, D), jnp.float32),
        mesh=mesh, scratch_types=scratch)
    def k1(msg_hbm, tgt_hbm, init_hbm, out_hbm, *s):
        body(msg_hbm, tgt_hbm, init_hbm, out_hbm, *s)

    return k1(msgs, tgts, init)


def _sum_partials_body(p_ref, o_ref):
    o_ref[...] = p_ref[0] + p_ref[1]


def _sum_partials(partials):
    BN = 2000
    return pl.pallas_call(
        _sum_partials_body,
        grid=(N // BN,),
        in_specs=[pl.BlockSpec((2, BN, D), lambda i: (0, i, 0))],
        out_specs=pl.BlockSpec((BN, D), lambda i: (i, 0)),
        out_shape=jax.ShapeDtypeStruct((N, D), jnp.float32),
    )(partials)


def kernel(node_states, adj_list_0, adj_list_1, adj_list_2, adj_list_3,
           W_0_0, W_0_1, W_1_0, W_1_1, W_2_0, W_2_1, W_3_0, W_3_1):
    adj = [adj_list_0, adj_list_1, adj_list_2, adj_list_3]
    w0cat = (jnp.stack([W_0_0, W_1_0, W_2_0, W_3_0])
             .reshape(T, 2, D, D).transpose(0, 2, 1, 3).reshape(T, D, 2 * D))
    w1s = jnp.stack([W_0_1, W_1_1, W_2_1, W_3_1])

    partials = None
    for types in ((0, 1), (2, 3)):
        t0 = types[0]
        TT = len(types)
        ab = _precompute(node_states, w0cat[t0:t0 + TT])
        table = ab.reshape(2 * TT * N, D)
        sidx = jnp.concatenate([adj[t][:, 0] + (t - t0) * N for t in types])
        tidx = jnp.concatenate(
            [adj[t][:, 1] + (TT + t - t0) * N for t in types])
        tgts = jnp.concatenate([adj[t][:, 1] for t in types])
        h = _gather_add(table, sidx, tidx)
        msgs = _mlp(h, w1s[t0:t0 + TT])
        partials = _scatter_add(msgs, tgts, partials)
    return _sum_partials(partials)


# speedup vs baseline: 5.6386x; 1.0439x over previous
import functools

import jax
import jax.numpy as jnp
from jax import lax
from jax.experimental import pallas as pl
from jax.experimental.pallas import tpu as pltpu
from jax.experimental.pallas import tpu_sc as plsc

N = 10000
D = 128
E = 80000
T = 4

NC = 2
NS = 16
NW = NC * NS

PREC = jax.lax.Precision.DEFAULT


def _precompute_body(ns_ref, w_ref, ab_ref):
    x = ns_ref[...]
    w = w_ref[0]
    y = jnp.dot(x, w, precision=PREC, preferred_element_type=jnp.float32)
    ab_ref[0, 0] = y[:, :D]
    ab_ref[1, 0] = y[:, D:]


def _precompute(ns, w0cat):
    TT = w0cat.shape[0]
    BN = 2000
    return pl.pallas_call(
        _precompute_body,
        grid=(TT, N // BN),
        in_specs=[
            pl.BlockSpec((BN, D), lambda t, i: (i, 0)),
            pl.BlockSpec((1, D, 2 * D), lambda t, i: (t, 0, 0)),
        ],
        out_specs=pl.BlockSpec((2, 1, BN, D), lambda t, i: (0, t, i, 0)),
        out_shape=jax.ShapeDtypeStruct((2, TT, N, D), jnp.float32),
    )(ns, w0cat)


def _gather_add(table, sidx, tidx):
    M = sidx.shape[0]
    per_w = M // NW
    C = 200
    steps = per_w // C
    assert steps >= 4 and per_w % C == 0
    mesh = plsc.VectorSubcoreMesh(core_axis_name="c", subcore_axis_name="s")

    @functools.partial(
        pl.kernel,
        out_type=jax.ShapeDtypeStruct((M, D), jnp.float32),
        mesh=mesh,
        scratch_types=(
            [pltpu.VMEM((C,), jnp.int32) for _ in range(4)]
            + [pltpu.VMEM((C, D), jnp.float32) for _ in range(4)]
            + [pltpu.SemaphoreType.DMA] * 10
        ),
    )
    def k(table_hbm, sidx_hbm, tidx_hbm, out_hbm, *scratch):
        isb = scratch[0:2]
        itb = scratch[2:4]
        ra = scratch[4:6]
        rb = scratch[6:8]
        ss = scratch[8:10]
        st = scratch[10:12]
        sa = scratch[12:14]
        sb = scratch[14:16]
        so = scratch[16:18]
        wid = lax.axis_index("s") * NC + lax.axis_index("c")
        base = wid * per_w

        def is_cp(kk, r):
            return pltpu.make_async_copy(
                sidx_hbm.at[pl.ds(base + kk * C, C)], isb[r], ss[r])

        def it_cp(kk, r):
            return pltpu.make_async_copy(
                tidx_hbm.at[pl.ds(base + kk * C, C)], itb[r], st[r])

        def gat_a(r):
            return pltpu.make_async_copy(table_hbm.at[isb[r]], ra[r], sa[r])

        def gat_b(r):
            return pltpu.make_async_copy(table_hbm.at[itb[r]], rb[r], sb[r])

        def add_rows(r):
            @plsc.parallel_loop(0, C, unroll=2)
            def _(i):
                for c in range(0, D, 16):
                    ra[r][i, pl.ds(c, 16)] += rb[r][i, pl.ds(c, 16)]

        def out_cp(kk, r):
            return pltpu.make_async_copy(
                ra[r], out_hbm.at[pl.ds(base + kk * C, C)], so[r])

        def ops(kk, r, first=False, second=False, last=False):
            rp = 1 - r
            if not (first or second):
                out_cp(kk - 2, r).wait()
            is_cp(kk, r).wait()
            it_cp(kk, r).wait()
            gat_a(r).start()
            gat_b(r).start()
            if not first:
                gat_a(rp).wait()
                gat_b(rp).wait()
                add_rows(rp)
                out_cp(kk - 1, rp).start()
                if not last:
                    is_cp(kk + 1, rp).start()
                    it_cp(kk + 1, rp).start()

        is_cp(0, 0).start()
        it_cp(0, 0).start()
        is_cp(1, 1).start()
        it_cp(1, 1).start()
        ops(0, 0, first=True)
        ops(1, 1, second=True, last=(steps == 2))

        even_lo = 2
        even_hi = even_lo + max(0, steps - even_lo - 2) // 2 * 2

        @pl.loop(even_lo, even_hi, step=2)
        def _(k0):
            for j in range(2):
                ops(k0 + j, j)

        for kk in range(even_hi, steps):
            ops(kk, kk % 2, last=(kk + 1 >= steps))

        r = (steps - 1) % 2
        gat_a(r).wait()
        gat_b(r).wait()
        add_rows(r)
        out_cp(steps - 1, r).start()
        out_cp(steps - 2, 1 - r).wait()
        out_cp(steps - 1, r).wait()

    return k(table, sidx, tidx)


def _mlp_body(h_ref, w_ref, m_ref):
    x = jnp.maximum(h_ref[...], 0.0)
    m = jnp.dot(x, w_ref[0], precision=PREC, preferred_element_type=jnp.float32)
    m_ref[...] = jnp.maximum(m, 0.0)


def _mlp(h, w1s):
    TT = w1s.shape[0]
    BE = 2000
    per_t = E // BE
    return pl.pallas_call(
        _mlp_body,
        grid=(TT, per_t),
        in_specs=[
            pl.BlockSpec((BE, D), lambda t, i: (t * per_t + i, 0)),
            pl.BlockSpec((1, D, D), lambda t, i: (t, 0, 0)),
        ],
        out_specs=pl.BlockSpec((BE, D), lambda t, i: (t * per_t + i, 0)),
        out_shape=jax.ShapeDtypeStruct((TT * E, D), jnp.float32),
    )(h, w1s)


def _scatter_add(msgs, tgts, init=None):
    M = msgs.shape[0]
    per_w = M // NW
    C = 192
    steps = per_w // C
    rem = per_w - steps * C
    assert steps >= 4 and steps % 2 == 0 and rem % 8 == 0 and rem < C
    rows_per_tile = 624
    tail = N - NS * rows_per_tile
    mesh = plsc.VectorSubcoreMesh(core_axis_name="c", subcore_axis_name="s")

    scratch = (
        [pltpu.VMEM((C,), jnp.int32) for _ in range(2)]
        + [pltpu.VMEM((max(rem, 8),), jnp.int32)]
        + [pltpu.VMEM((C, D), jnp.float32) for _ in range(2)]
        + [pltpu.SemaphoreType.DMA] * 6
        + [pltpu.VMEM_SHARED((N, D), jnp.float32)]
    )

    def body(msg_hbm, tgt_hbm, init_hbm, out_hbm,
             ib0, ib1, ibt, mb0, mb1, si0, si1, sm0, sm1, sa0, sa1, acc):
        ib = (ib0, ib1)
        mb = (mb0, mb1)
        si = (si0, si1)
        sm = (sm0, sm1)
        sa = (sa0, sa1)
        cid = lax.axis_index("c")
        sid = lax.axis_index("s")
        wid = sid * NC + cid
        base = wid * per_w

        if init_hbm is None:
            zvec = jnp.zeros((16,), jnp.float32)

            @plsc.parallel_loop(0, C)
            def _(i):
                for c in range(0, D, 16):
                    mb0[i, pl.ds(c, 16)] = zvec

            rows = N // NS
            nfull = rows // C
            r0z = sid * rows
            for j in range(nfull):
                pltpu.sync_copy(mb0, acc.at[pl.ds(r0z + j * C, C)])
            pltpu.sync_copy(mb0.at[pl.ds(0, rows - nfull * C)],
                            acc.at[pl.ds(r0z + nfull * C, rows - nfull * C)])
        else:
            @pl.when(sid == 0)
            def _():
                pltpu.sync_copy(init_hbm.at[cid], acc)

        plsc.subcore_barrier()

        def idx_cp(kk, r):
            return pltpu.make_async_copy(
                tgt_hbm.at[pl.ds(base + kk * C, C)], ib[r], si[r])

        def msg_cp(kk, r):
            return pltpu.make_async_copy(
                msg_hbm.at[pl.ds(base + kk * C, C)], mb[r], sm[r])

        def sca_start(r):
            pltpu.async_copy(mb[r], acc.at[ib[r]], sa[r], add=True)

        def sca_wait(r):
            pltpu.make_async_copy(mb[r], acc.at[ib[r]], sa[r]).wait()

        def ops(kk, r, first=False, last=False):
            if not first:
                sca_wait(1 - r)
            if not last:
                idx_cp(kk + 1, 1 - r).start()
                msg_cp(kk + 1, 1 - r).start()
            idx_cp(kk, r).wait()
            msg_cp(kk, r).wait()
            sca_start(r)

        idx_cp(0, 0).start()
        msg_cp(0, 0).start()
        ops(0, 0, first=True)
        ops(1, 1)

        @pl.loop(2, steps, step=2)
        def _(k0):
            for j in range(2):
                kk = k0 + j
                sca_wait(1 - j)

                def _pref(kk=kk, j=j):
                    idx_cp(kk + 1, 1 - j).start()
                    msg_cp(kk + 1, 1 - j).start()

                pl.when(kk + 1 < steps)(_pref)
                idx_cp(kk, j).wait()
                msg_cp(kk, j).wait()
                sca_start(j)

        sca_wait((steps - 1) % 2)
        if rem:
            rr = (steps - 1) % 2
            pltpu.sync_copy(tgt_hbm.at[pl.ds(base + steps * C, rem)], ibt)
            pltpu.sync_copy(msg_hbm.at[pl.ds(base + steps * C, rem)],
                            mb[rr].at[pl.ds(0, rem)])
            pltpu.sync_copy(mb[rr].at[pl.ds(0, rem)], acc.at[ibt], add=True)

        plsc.subcore_barrier()
        r0 = sid * rows_per_tile
        pltpu.sync_copy(acc.at[pl.ds(r0, rows_per_tile)],
                        out_hbm.at[cid, pl.ds(r0, rows_per_tile)])

        @pl.when(sid == 0)
        def _():
            t0 = NS * rows_per_tile
            pltpu.sync_copy(acc.at[pl.ds(t0, tail)],
                            out_hbm.at[cid, pl.ds(t0, tail)])

    if init is None:
        @functools.partial(
            pl.kernel,
            out_type=jax.ShapeDtypeStruct((NC, N, D), jnp.float32),
            mesh=mesh, scratch_types=scratch)
        def k0(msg_hbm, tgt_hbm, out_hbm, *s):
            body(msg_hbm, tgt_hbm, None, out_hbm, *s)

        return k0(msgs, tgts)

    @functools.partial(
        pl.kernel,
        out_type=jax.ShapeDtypeStruct((NC, N, D), jnp.float32),
        mesh=mesh, scratch_types=scratch)
    def k1(msg_hbm, tgt_hbm, init_hbm, out_hbm, *s):
        body(msg_hbm, tgt_hbm, init_hbm, out_hbm, *s)

    return k1(msgs, tgts, init)


def _sum_partials_body(p_ref, o_ref):
    o_ref[...] = p_ref[0] + p_ref[1]


def _sum_partials(partials):
    BN = 2000
    return pl.pallas_call(
        _sum_partials_body,
        grid=(N // BN,),
        in_specs=[pl.BlockSpec((2, BN, D), lambda i: (0, i, 0))],
        out_specs=pl.BlockSpec((BN, D), lambda i: (i, 0)),
        out_shape=jax.ShapeDtypeStruct((N, D), jnp.float32),
    )(partials)


def kernel(node_states, adj_list_0, adj_list_1, adj_list_2, adj_list_3,
           W_0_0, W_0_1, W_1_0, W_1_1, W_2_0, W_2_1, W_3_0, W_3_1):
    adj = [adj_list_0, adj_list_1, adj_list_2, adj_list_3]
    w0cat = (jnp.stack([W_0_0, W_1_0, W_2_0, W_3_0])
             .reshape(T, 2, D, D).transpose(0, 2, 1, 3).reshape(T, D, 2 * D))
    w1s = jnp.stack([W_0_1, W_1_1, W_2_1, W_3_1])

    partials = None
    for types in ((0, 1), (2, 3)):
        t0 = types[0]
        TT = len(types)
        ab = _precompute(node_states, w0cat[t0:t0 + TT])
        table = ab.reshape(2 * TT * N, D)
        sidx = jnp.concatenate([adj[t][:, 0] + (t - t0) * N for t in types])
        tidx = jnp.concatenate(
            [adj[t][:, 1] + (TT + t - t0) * N for t in types])
        tgts = jnp.concatenate([adj[t][:, 1] for t in types])
        h = _gather_add(table, sidx, tidx)
        msgs = _mlp(h, w1s[t0:t0 + TT])
        partials = _scatter_add(msgs, tgts, partials)
    return _sum_partials(partials)


# speedup vs baseline: 5.7937x; 1.0275x over previous
import functools

import jax
import jax.numpy as jnp
from jax import lax
from jax.experimental import pallas as pl
from jax.experimental.pallas import tpu as pltpu
from jax.experimental.pallas import tpu_sc as plsc

N = 10000
D = 128
E = 80000
T = 4

NC = 2
NS = 16
NW = NC * NS

PREC = jax.lax.Precision.DEFAULT


def _precompute_body(ns_ref, w_ref, ab_ref):
    x = ns_ref[...]
    w = w_ref[0]
    y = jnp.dot(x, w, precision=PREC, preferred_element_type=jnp.float32)
    ab_ref[0, 0] = y[:, :D]
    ab_ref[1, 0] = y[:, D:]


def _precompute(ns, w0cat):
    TT = w0cat.shape[0]
    BN = 2000
    return pl.pallas_call(
        _precompute_body,
        grid=(TT, N // BN),
        in_specs=[
            pl.BlockSpec((BN, D), lambda t, i: (i, 0)),
            pl.BlockSpec((1, D, 2 * D), lambda t, i: (t, 0, 0)),
        ],
        out_specs=pl.BlockSpec((2, 1, BN, D), lambda t, i: (0, t, i, 0)),
        out_shape=jax.ShapeDtypeStruct((2, TT, N, D), jnp.float32),
    )(ns, w0cat)


def _gather_add(table, sidx, tidx):
    M = sidx.shape[0]
    per_w = M // NW
    C = 200
    steps = per_w // C
    assert steps >= 4 and per_w % C == 0
    mesh = plsc.VectorSubcoreMesh(core_axis_name="c", subcore_axis_name="s")

    @functools.partial(
        pl.kernel,
        out_type=jax.ShapeDtypeStruct((M, D), jnp.float32),
        mesh=mesh,
        scratch_types=(
            [pltpu.VMEM((C,), jnp.int32) for _ in range(4)]
            + [pltpu.VMEM((C, D), jnp.float32) for _ in range(4)]
            + [pltpu.SemaphoreType.DMA] * 10
        ),
    )
    def k(table_hbm, sidx_hbm, tidx_hbm, out_hbm, *scratch):
        isb = scratch[0:2]
        itb = scratch[2:4]
        ra = scratch[4:6]
        rb = scratch[6:8]
        ss = scratch[8:10]
        st = scratch[10:12]
        sa = scratch[12:14]
        sb = scratch[14:16]
        so = scratch[16:18]
        wid = lax.axis_index("s") * NC + lax.axis_index("c")
        base = wid * per_w

        def is_cp(kk, r):
            return pltpu.make_async_copy(
                sidx_hbm.at[pl.ds(base + kk * C, C)], isb[r], ss[r])

        def it_cp(kk, r):
            return pltpu.make_async_copy(
                tidx_hbm.at[pl.ds(base + kk * C, C)], itb[r], st[r])

        def gat_a(r):
            return pltpu.make_async_copy(table_hbm.at[isb[r]], ra[r], sa[r])

        def gat_b(r):
            return pltpu.make_async_copy(table_hbm.at[itb[r]], rb[r], sb[r])

        def add_rows(r):
            @plsc.parallel_loop(0, C, unroll=2)
            def _(i):
                for c in range(0, D, 16):
                    ra[r][i, pl.ds(c, 16)] += rb[r][i, pl.ds(c, 16)]

        def out_cp(kk, r):
            return pltpu.make_async_copy(
                ra[r], out_hbm.at[pl.ds(base + kk * C, C)], so[r])

        def ops(kk, r, first=False, second=False, last=False):
            rp = 1 - r
            if not (first or second):
                out_cp(kk - 2, r).wait()
            is_cp(kk, r).wait()
            it_cp(kk, r).wait()
            gat_a(r).start()
            gat_b(r).start()
            if not first:
                gat_a(rp).wait()
                gat_b(rp).wait()
                add_rows(rp)
                out_cp(kk - 1, rp).start()
                if not last:
                    is_cp(kk + 1, rp).start()
                    it_cp(kk + 1, rp).start()

        is_cp(0, 0).start()
        it_cp(0, 0).start()
        is_cp(1, 1).start()
        it_cp(1, 1).start()
        ops(0, 0, first=True)
        ops(1, 1, second=True, last=(steps == 2))

        even_lo = 2
        even_hi = even_lo + max(0, steps - even_lo - 2) // 2 * 2

        @pl.loop(even_lo, even_hi, step=2)
        def _(k0):
            for j in range(2):
                ops(k0 + j, j)

        for kk in range(even_hi, steps):
            ops(kk, kk % 2, last=(kk + 1 >= steps))

        r = (steps - 1) % 2
        gat_a(r).wait()
        gat_b(r).wait()
        add_rows(r)
        out_cp(steps - 1, r).start()
        out_cp(steps - 2, 1 - r).wait()
        out_cp(steps - 1, r).wait()

    return k(table, sidx, tidx)


def _mlp_body(h_ref, w_ref, m_ref):
    x = jnp.maximum(h_ref[...], 0.0)
    m = jnp.dot(x, w_ref[0], precision=PREC, preferred_element_type=jnp.float32)
    m_ref[...] = jnp.maximum(m, 0.0)


def _mlp(h, w1s):
    TT = w1s.shape[0]
    BE = 4000
    per_t = E // BE
    return pl.pallas_call(
        _mlp_body,
        grid=(TT, per_t),
        in_specs=[
            pl.BlockSpec((BE, D), lambda t, i: (t * per_t + i, 0)),
            pl.BlockSpec((1, D, D), lambda t, i: (t, 0, 0)),
        ],
        out_specs=pl.BlockSpec((BE, D), lambda t, i: (t * per_t + i, 0)),
        out_shape=jax.ShapeDtypeStruct((TT * E, D), jnp.float32),
    )(h, w1s)


def _scatter_add(msgs, tgts, init=None):
    M = msgs.shape[0]
    per_w = M // NW
    C = 192
    steps = per_w // C
    rem = per_w - steps * C
    assert steps >= 4 and steps % 2 == 0 and rem % 8 == 0 and rem < C
    rows_per_tile = 624
    tail = N - NS * rows_per_tile
    mesh = plsc.VectorSubcoreMesh(core_axis_name="c", subcore_axis_name="s")

    scratch = (
        [pltpu.VMEM((C,), jnp.int32) for _ in range(2)]
        + [pltpu.VMEM((max(rem, 8),), jnp.int32)]
        + [pltpu.VMEM((C, D), jnp.float32) for _ in range(2)]
        + [pltpu.SemaphoreType.DMA] * 6
        + [pltpu.VMEM_SHARED((N, D), jnp.float32)]
    )

    def body(msg_hbm, tgt_hbm, init_hbm, out_hbm,
             ib0, ib1, ibt, mb0, mb1, si0, si1, sm0, sm1, sa0, sa1, acc):
        ib = (ib0, ib1)
        mb = (mb0, mb1)
        si = (si0, si1)
        sm = (sm0, sm1)
        sa = (sa0, sa1)
        cid = lax.axis_index("c")
        sid = lax.axis_index("s")
        wid = sid * NC + cid
        base = wid * per_w

        if init_hbm is None:
            zvec = jnp.zeros((16,), jnp.float32)

            @plsc.parallel_loop(0, C)
            def _(i):
                for c in range(0, D, 16):
                    mb0[i, pl.ds(c, 16)] = zvec

            rows = N // NS
            nfull = rows // C
            r0z = sid * rows
            for j in range(nfull):
                pltpu.sync_copy(mb0, acc.at[pl.ds(r0z + j * C, C)])
            pltpu.sync_copy(mb0.at[pl.ds(0, rows - nfull * C)],
                            acc.at[pl.ds(r0z + nfull * C, rows - nfull * C)])
        else:
            @pl.when(sid == 0)
            def _():
                pltpu.sync_copy(init_hbm.at[cid], acc)

        plsc.subcore_barrier()

        def idx_cp(kk, r):
            return pltpu.make_async_copy(
                tgt_hbm.at[pl.ds(base + kk * C, C)], ib[r], si[r])

        def msg_cp(kk, r):
            return pltpu.make_async_copy(
                msg_hbm.at[pl.ds(base + kk * C, C)], mb[r], sm[r])

        def sca_start(r):
            pltpu.async_copy(mb[r], acc.at[ib[r]], sa[r], add=True)

        def sca_wait(r):
            pltpu.make_async_copy(mb[r], acc.at[ib[r]], sa[r]).wait()

        def ops(kk, r, first=False, last=False):
            if not first:
                sca_wait(1 - r)
            if not last:
                idx_cp(kk + 1, 1 - r).start()
                msg_cp(kk + 1, 1 - r).start()
            idx_cp(kk, r).wait()
            msg_cp(kk, r).wait()
            sca_start(r)

        idx_cp(0, 0).start()
        msg_cp(0, 0).start()
        ops(0, 0, first=True)
        ops(1, 1)

        @pl.loop(2, steps, step=2)
        def _(k0):
            for j in range(2):
                kk = k0 + j
                sca_wait(1 - j)

                def _pref(kk=kk, j=j):
                    idx_cp(kk + 1, 1 - j).start()
                    msg_cp(kk + 1, 1 - j).start()

                pl.when(kk + 1 < steps)(_pref)
                idx_cp(kk, j).wait()
                msg_cp(kk, j).wait()
                sca_start(j)

        sca_wait((steps - 1) % 2)
        if rem:
            rr = (steps - 1) % 2
            pltpu.sync_copy(tgt_hbm.at[pl.ds(base + steps * C, rem)], ibt)
            pltpu.sync_copy(msg_hbm.at[pl.ds(base + steps * C, rem)],
                            mb[rr].at[pl.ds(0, rem)])
            pltpu.sync_copy(mb[rr].at[pl.ds(0, rem)], acc.at[ibt], add=True)

        plsc.subcore_barrier()
        r0 = sid * rows_per_tile
        pltpu.sync_copy(acc.at[pl.ds(r0, rows_per_tile)],
                        out_hbm.at[cid, pl.ds(r0, rows_per_tile)])

        @pl.when(sid == 0)
        def _():
            t0 = NS * rows_per_tile
            pltpu.sync_copy(acc.at[pl.ds(t0, tail)],
                            out_hbm.at[cid, pl.ds(t0, tail)])

    if init is None:
        @functools.partial(
            pl.kernel,
            out_type=jax.ShapeDtypeStruct((NC, N, D), jnp.float32),
            mesh=mesh, scratch_types=scratch)
        def k0(msg_hbm, tgt_hbm, out_hbm, *s):
            body(msg_hbm, tgt_hbm, None, out_hbm, *s)

        return k0(msgs, tgts)

    @functools.partial(
        pl.kernel,
        out_type=jax.ShapeDtypeStruct((NC, N, D), jnp.float32),
        mesh=mesh, scratch_types=scratch)
    def k1(msg_hbm, tgt_hbm, init_hbm, out_hbm, *s):
        body(msg_hbm, tgt_hbm, init_hbm, out_hbm, *s)

    return k1(msgs, tgts, init)


def _sum_partials_body(p_ref, o_ref):
    o_ref[...] = p_ref[0] + p_ref[1]


def _sum_partials(partials):
    BN = 2000
    return pl.pallas_call(
        _sum_partials_body,
        grid=(N // BN,),
        in_specs=[pl.BlockSpec((2, BN, D), lambda i: (0, i, 0))],
        out_specs=pl.BlockSpec((BN, D), lambda i: (i, 0)),
        out_shape=jax.ShapeDtypeStruct((N, D), jnp.float32),
    )(partials)


def kernel(node_states, adj_list_0, adj_list_1, adj_list_2, adj_list_3,
           W_0_0, W_0_1, W_1_0, W_1_1, W_2_0, W_2_1, W_3_0, W_3_1):
    adj = [adj_list_0, adj_list_1, adj_list_2, adj_list_3]
    w0cat = (jnp.stack([W_0_0, W_1_0, W_2_0, W_3_0])
             .reshape(T, 2, D, D).transpose(0, 2, 1, 3).reshape(T, D, 2 * D))
    w1s = jnp.stack([W_0_1, W_1_1, W_2_1, W_3_1])

    partials = None
    for types in ((0, 1), (2, 3)):
        t0 = types[0]
        TT = len(types)
        ab = _precompute(node_states, w0cat[t0:t0 + TT])
        table = ab.reshape(2 * TT * N, D)
        sidx = jnp.concatenate([adj[t][:, 0] + (t - t0) * N for t in types])
        tidx = jnp.concatenate(
            [adj[t][:, 1] + (TT + t - t0) * N for t in types])
        tgts = jnp.concatenate([adj[t][:, 1] for t in types])
        h = _gather_add(table, sidx, tidx)
        msgs = _mlp(h, w1s[t0:t0 + TT])
        partials = _scatter_add(msgs, tgts, partials)
    return _sum_partials(partials)
